# Initial kernel scaffold; baseline (speedup 1.0000x reference)
import jax
import jax.numpy as jnp
from jax.experimental import pallas as pl


def kernel(x, weight):
    raise NotImplementedError("write your pallas kernel here")


# speedup vs baseline: 1.3905x; 1.3905x over previous
import jax
import jax.numpy as jnp
from jax.experimental import pallas as pl
from jax.experimental.pallas import tpu as pltpu

_CONTRACT_LAST = (((1,), (1,)), ((), ()))


def _mm_kernel(x_ref, w_ref, o_ref):
    o_ref[...] = jax.lax.dot_general(
        x_ref[...], w_ref[...],
        dimension_numbers=_CONTRACT_LAST,
        preferred_element_type=jnp.float32,
    )


def _linear(x2d, w, tm, tn):
    M, K = x2d.shape
    N = w.shape[0]
    grid = (M // tm, N // tn)
    out = pl.pallas_call(
        _mm_kernel,
        out_shape=jax.ShapeDtypeStruct((M, N), jnp.float32),
        grid=grid,
        in_specs=[
            pl.BlockSpec((tm, K), lambda i, j: (i, 0)),
            pl.BlockSpec((tn, K), lambda i, j: (j, 0)),
        ],
        out_specs=pl.BlockSpec((tm, tn), lambda i, j: (i, j)),
        compiler_params=pltpu.CompilerParams(
            dimension_semantics=("parallel", "parallel"),
            vmem_limit_bytes=56 << 20,
        ),
        cost_estimate=pl.CostEstimate(
            flops=2 * M * N * K,
            bytes_accessed=(M * K + N * K) * 2 + M * N * 4,
            transcendentals=0,
        ),
    )(x2d, w)
    return out


def kernel(x, weight):
    orig_lead = x.shape[:-1]
    K = x.shape[-1]
    x2d = x.reshape(-1, K).astype(jnp.bfloat16)
    w = weight.astype(jnp.bfloat16)
    out = _linear(x2d, w, tm=1024, tn=1024)
    return out.reshape(*orig_lead, weight.shape[0])


# speedup vs baseline: 1.7412x; 1.2522x over previous
import jax
import jax.numpy as jnp
from jax.experimental import pallas as pl
from jax.experimental.pallas import tpu as pltpu

_CONTRACT_LAST = (((1,), (1,)), ((), ()))


def _mm_kernel(x_ref, w_ref, o_ref):
    o_ref[...] = jax.lax.dot_general(
        x_ref[...], w_ref[...],
        dimension_numbers=_CONTRACT_LAST,
        preferred_element_type=jnp.float32,
    )


def _mm_cast_accum_kernel(x_ref, w_ref, o_ref):
    @pl.when(pl.program_id(2) == 0)
    def _():
        o_ref[...] = jnp.zeros_like(o_ref)

    o_ref[...] += jax.lax.dot_general(
        x_ref[...].astype(jnp.bfloat16), w_ref[...].astype(jnp.bfloat16),
        dimension_numbers=_CONTRACT_LAST,
        preferred_element_type=jnp.float32,
    )


def _linear_fused(x2d, w, tm, tn, tk):
    M, K = x2d.shape
    N = w.shape[0]
    grid = (M // tm, N // tn, K // tk)
    out = pl.pallas_call(
        _mm_cast_accum_kernel,
        out_shape=jax.ShapeDtypeStruct((M, N), jnp.float32),
        grid=grid,
        in_specs=[
            pl.BlockSpec((tm, tk), lambda i, j, k: (i, k)),
            pl.BlockSpec((tn, tk), lambda i, j, k: (j, k)),
        ],
        out_specs=pl.BlockSpec((tm, tn), lambda i, j, k: (i, j)),
        compiler_params=pltpu.CompilerParams(
            dimension_semantics=("parallel", "parallel", "arbitrary"),
            vmem_limit_bytes=60 << 20,
        ),
        cost_estimate=pl.CostEstimate(
            flops=2 * M * N * K,
            bytes_accessed=(M * K + N * K) * 4 + M * N * 4,
            transcendentals=0,
        ),
    )(x2d, w)
    return out


def _linear(x2d, w, tm, tn):
    M, K = x2d.shape
    N = w.shape[0]
    grid = (M // tm, N // tn)
    out = pl.pallas_call(
        _mm_kernel,
        out_shape=jax.ShapeDtypeStruct((M, N), jnp.float32),
        grid=grid,
        in_specs=[
            pl.BlockSpec((tm, K), lambda i, j: (i, 0)),
            pl.BlockSpec((tn, K), lambda i, j: (j, 0)),
        ],
        out_specs=pl.BlockSpec((tm, tn), lambda i, j: (i, j)),
        compiler_params=pltpu.CompilerParams(
            dimension_semantics=("parallel", "parallel"),
            vmem_limit_bytes=56 << 20,
        ),
        cost_estimate=pl.CostEstimate(
            flops=2 * M * N * K,
            bytes_accessed=(M * K + N * K) * 2 + M * N * 4,
            transcendentals=0,
        ),
    )(x2d, w)
    return out


def kernel(x, weight):
    orig_lead = x.shape[:-1]
    K = x.shape[-1]
    x2d = x.reshape(-1, K)
    out = _linear_fused(x2d, weight, tm=2048, tn=2048, tk=512)
    return out.reshape(*orig_lead, weight.shape[0])
